# baseline (speedup 1.0000x reference)
import functools

import jax
import jax.numpy as jnp
from jax import lax
from jax.experimental import pallas as pl
from jax.experimental.pallas import tpu as pltpu
from jax.experimental.pallas import tpu_sc as plsc


def _make_gather(R_out, R_src, Hdim, Ndim):
    info = plsc.get_sparse_core_info()
    NC, NS, L = info.num_cores, info.num_subcores, info.num_lanes
    NW = NC * NS
    assert Ndim == L, "design assumes batch dim == lane count"
    rows_per_w = R_out // NW
    assert rows_per_w * NW == R_out
    K = 64
    NBUF = 2
    n_chunks = rows_per_w // K
    assert n_chunks * K == rows_per_w
    n_groups = n_chunks // NBUF
    assert n_groups * NBUF == n_chunks

    mesh = plsc.VectorSubcoreMesh(core_axis_name="c", subcore_axis_name="s")

    @functools.partial(
        pl.kernel,
        out_type=jax.ShapeDtypeStruct((R_out, Hdim), jnp.float32),
        mesh=mesh,
        scratch_types=[
            pltpu.VMEM((rows_per_w,), jnp.int32),
            pltpu.VMEM((NBUF, K, Hdim), jnp.float32),
            pltpu.SemaphoreType.DMA((NBUF,)),
            pltpu.SemaphoreType.DMA((NBUF,)),
        ],
    )
    def body(x_hbm, t_hbm, out_hbm, idx_v, rows_v, gsem, wsem):
        wid = lax.axis_index("s") * NC + lax.axis_index("c")
        base = wid * rows_per_w
        pltpu.sync_copy(t_hbm.at[pl.ds(base, rows_per_w)], idx_v)
        lane = lax.iota(jnp.int32, L)

        def cvt(i, carry):
            off = pl.multiple_of(i * L, L)
            tv = idx_v[pl.ds(off, L)]
            idx_v[pl.ds(off, L)] = (tv - 1) * Ndim + lane
            return carry

        lax.fori_loop(0, rows_per_w // L, cvt, 0)

        def gdesc(c, b):
            coff = pl.multiple_of(c * K, K)
            return pltpu.make_async_copy(
                x_hbm.at[idx_v.at[pl.ds(coff, K)]], rows_v.at[b], gsem.at[b]
            )

        def wdesc(c, b):
            coff = pl.multiple_of(c * K, K)
            return pltpu.make_async_copy(
                rows_v.at[b], out_hbm.at[pl.ds(base + coff, K)], wsem.at[b]
            )

        def group(g, carry):
            for b in range(NBUF):
                c = g * NBUF + b

                @pl.when(g > 0)
                def _():
                    wdesc(c - NBUF, b).wait()

                gdesc(c, b).start()
            for b in range(NBUF):
                c = g * NBUF + b
                gdesc(c, b).wait()
                wdesc(c, b).start()
            return carry

        lax.fori_loop(0, n_groups, group, 0)
        for b in range(NBUF):
            wdesc(n_chunks - NBUF + b, b).wait()

    return body


def kernel(x, targets, lengths):
    Tdim, Ndim, Hdim = x.shape
    Tout = targets.shape[0]
    xflat = x.reshape(Tdim * Ndim, Hdim)
    tflat = targets.reshape(Tout * Ndim)
    gather = _make_gather(Tout * Ndim, Tdim * Ndim, Hdim, Ndim)
    out = gather(xflat, tflat)
    return (out.reshape(Tout, Ndim, Hdim), lengths)


# speedup vs baseline: 1.0416x; 1.0416x over previous
import functools

import jax
import jax.numpy as jnp
from jax import lax
from jax.experimental import pallas as pl
from jax.experimental.pallas import tpu as pltpu
from jax.experimental.pallas import tpu_sc as plsc


def _make_gather(R_out, R_src, Hdim, Ndim):
    info = plsc.get_sparse_core_info()
    NC, NS, L = info.num_cores, info.num_subcores, info.num_lanes
    NW = NC * NS
    assert Ndim == L, "design assumes batch dim == lane count"
    rows_per_w = R_out // NW
    assert rows_per_w * NW == R_out
    K = 16
    NBUF = 8
    n_chunks = rows_per_w // K
    assert n_chunks * K == rows_per_w
    n_groups = n_chunks // NBUF
    assert n_groups * NBUF == n_chunks

    mesh = plsc.VectorSubcoreMesh(core_axis_name="c", subcore_axis_name="s")

    @functools.partial(
        pl.kernel,
        out_type=jax.ShapeDtypeStruct((R_out, Hdim), jnp.float32),
        mesh=mesh,
        scratch_types=[
            pltpu.VMEM((rows_per_w,), jnp.int32),
            pltpu.VMEM((NBUF, K, Hdim), jnp.float32),
            pltpu.SemaphoreType.DMA((NBUF,)),
            pltpu.SemaphoreType.DMA((NBUF,)),
        ],
    )
    def body(x_hbm, t_hbm, out_hbm, idx_v, rows_v, gsem, wsem):
        wid = lax.axis_index("s") * NC + lax.axis_index("c")
        base = wid * rows_per_w
        pltpu.sync_copy(t_hbm.at[pl.ds(base, rows_per_w)], idx_v)
        lane = lax.iota(jnp.int32, L)

        def cvt(i, carry):
            off = pl.multiple_of(i * L, L)
            tv = idx_v[pl.ds(off, L)]
            idx_v[pl.ds(off, L)] = (tv - 1) * Ndim + lane
            return carry

        lax.fori_loop(0, rows_per_w // L, cvt, 0)

        def gdesc(c, b):
            coff = pl.multiple_of(c * K, K)
            return pltpu.make_async_copy(
                x_hbm.at[idx_v.at[pl.ds(coff, K)]], rows_v.at[b], gsem.at[b]
            )

        def wdesc(c, b):
            coff = pl.multiple_of(c * K, K)
            return pltpu.make_async_copy(
                rows_v.at[b], out_hbm.at[pl.ds(base + coff, K)], wsem.at[b]
            )

        def group(g, carry):
            for b in range(NBUF):
                c = g * NBUF + b

                @pl.when(g > 0)
                def _():
                    wdesc(c - NBUF, b).wait()

                gdesc(c, b).start()
            for b in range(NBUF):
                c = g * NBUF + b
                gdesc(c, b).wait()
                wdesc(c, b).start()
            return carry

        lax.fori_loop(0, n_groups, group, 0)
        for b in range(NBUF):
            wdesc(n_chunks - NBUF + b, b).wait()

    return body


def kernel(x, targets, lengths):
    Tdim, Ndim, Hdim = x.shape
    Tout = targets.shape[0]
    xflat = x.reshape(Tdim * Ndim, Hdim)
    tflat = targets.reshape(Tout * Ndim)
    gather = _make_gather(Tout * Ndim, Tdim * Ndim, Hdim, Ndim)
    out = gather(xflat, tflat)
    return (out.reshape(Tout, Ndim, Hdim), lengths)


# speedup vs baseline: 1.0475x; 1.0056x over previous
import functools

import jax
import jax.numpy as jnp
from jax import lax
from jax.experimental import pallas as pl
from jax.experimental.pallas import tpu as pltpu
from jax.experimental.pallas import tpu_sc as plsc


def _make_gather(R_out, R_src, Hdim, Ndim):
    info = plsc.get_sparse_core_info()
    NC, NS, L = info.num_cores, info.num_subcores, info.num_lanes
    NW = NC * NS
    assert Ndim == L, "design assumes batch dim == lane count"
    rows_per_w = R_out // NW
    assert rows_per_w * NW == R_out
    K = 16
    NBUF = 8
    n_chunks = rows_per_w // K
    assert n_chunks * K == rows_per_w
    n_groups = n_chunks // NBUF
    assert n_groups * NBUF == n_chunks

    mesh = plsc.VectorSubcoreMesh(core_axis_name="c", subcore_axis_name="s")

    @functools.partial(
        pl.kernel,
        out_type=jax.ShapeDtypeStruct((R_out, Hdim), jnp.float32),
        mesh=mesh,
        scratch_types=[
            pltpu.VMEM((rows_per_w,), jnp.int32),
            pltpu.VMEM((NBUF, K, Hdim), jnp.float32),
            pltpu.SemaphoreType.DMA((NBUF,)),
            pltpu.SemaphoreType.DMA((NBUF,)),
        ],
    )
    def body(x_hbm, t_hbm, out_hbm, idx_v, rows_v, gsem, wsem):
        wid = lax.axis_index("c") * NS + lax.axis_index("s")
        base = wid * rows_per_w
        pltpu.sync_copy(t_hbm.at[pl.ds(base, rows_per_w)], idx_v)
        lane = lax.iota(jnp.int32, L)

        def cvt(i, carry):
            off = pl.multiple_of(i * L, L)
            tv = idx_v[pl.ds(off, L)]
            idx_v[pl.ds(off, L)] = (tv - 1) * Ndim + lane
            return carry

        lax.fori_loop(0, rows_per_w // L, cvt, 0)

        def gdesc(c, b):
            coff = pl.multiple_of(c * K, K)
            return pltpu.make_async_copy(
                x_hbm.at[idx_v.at[pl.ds(coff, K)]], rows_v.at[b], gsem.at[b]
            )

        def wdesc(c, b):
            coff = pl.multiple_of(c * K, K)
            return pltpu.make_async_copy(
                rows_v.at[b], out_hbm.at[pl.ds(base + coff, K)], wsem.at[b]
            )

        def group(g, carry):
            for b in range(NBUF):
                c = g * NBUF + b

                @pl.when(g > 0)
                def _():
                    wdesc(c - NBUF, b).wait()

                gdesc(c, b).start()
            for b in range(NBUF):
                c = g * NBUF + b
                gdesc(c, b).wait()
                wdesc(c, b).start()
            return carry

        lax.fori_loop(0, n_groups, group, 0)
        for b in range(NBUF):
            wdesc(n_chunks - NBUF + b, b).wait()

    return body


def kernel(x, targets, lengths):
    Tdim, Ndim, Hdim = x.shape
    Tout = targets.shape[0]
    xflat = x.reshape(Tdim * Ndim, Hdim)
    tflat = targets.reshape(Tout * Ndim)
    gather = _make_gather(Tout * Ndim, Tdim * Ndim, Hdim, Ndim)
    out = gather(xflat, tflat)
    return (out.reshape(Tout, Ndim, Hdim), lengths)
